# baseline (speedup 1.0000x reference)
import jax
import jax.numpy as jnp
from jax.experimental import pallas as pl

B, P, N = 8, 512, 128
IN_DIM, HID, OUT_DIM = 3, 64, 60
AUG = IN_DIM + 1
G = 128


def _body(x_ref, w1a_ref, w2_ref, b2_ref, out_ref):
    x = x_ref[...].astype(jnp.bfloat16)
    xa = jnp.concatenate(
        [x, jnp.ones((1, G * N), jnp.bfloat16)], axis=0)
    ht = jnp.dot(w1a_ref[...], xa,
                 preferred_element_type=jnp.float32)
    ht = jnp.maximum(ht.astype(jnp.bfloat16), 0)
    o = jax.lax.dot_general(
        ht, w2_ref[...],
        (((0,), (0,)), ((), ())),
        preferred_element_type=jnp.float32)
    o = o.astype(jnp.bfloat16).reshape(G, N, OUT_DIM)
    half = N // 2
    while half >= 8:
        o = jnp.maximum(o[:, :half, :], o[:, half:, :])
        half //= 2
    m = jnp.max(o, axis=1).astype(jnp.float32)
    out_ref[...] = m + b2_ref[...]


def kernel(point_groups, W1, b1, W2, b2):
    num_patches = B * P
    xt = jnp.swapaxes(point_groups, -1, -2)
    xt = xt.reshape(num_patches, IN_DIM, N)
    xt = jnp.transpose(xt, (1, 0, 2)).reshape(IN_DIM, num_patches * N)
    w1a = jnp.concatenate([W1, b1.reshape(1, HID)], axis=0).T
    grid = (num_patches // G,)
    out = pl.pallas_call(
        _body,
        grid=grid,
        in_specs=[
            pl.BlockSpec((IN_DIM, G * N), lambda i: (0, i)),
            pl.BlockSpec((HID, AUG), lambda i: (0, 0)),
            pl.BlockSpec((HID, OUT_DIM), lambda i: (0, 0)),
            pl.BlockSpec((1, OUT_DIM), lambda i: (0, 0)),
        ],
        out_specs=pl.BlockSpec((G, OUT_DIM), lambda i: (i, 0)),
        out_shape=jax.ShapeDtypeStruct((num_patches, OUT_DIM), jnp.float32),
    )(xt, w1a.astype(jnp.bfloat16), W2.astype(jnp.bfloat16),
      b2.reshape(1, OUT_DIM))
    return out.reshape(B, P, OUT_DIM)


# speedup vs baseline: 1.1080x; 1.1080x over previous
import jax
import jax.numpy as jnp
from jax.experimental import pallas as pl

B, P, N = 8, 512, 128
IN_DIM, HID, OUT_DIM = 3, 64, 60
AUG = IN_DIM + 1
G = 256


def _body(x_ref, w1a_ref, w2_ref, b2_ref, out_ref):
    x = x_ref[...]
    ht = jnp.dot(w1a_ref[...], x,
                 preferred_element_type=jnp.float32)
    ht = jnp.maximum(ht.astype(jnp.bfloat16), 0)
    o = jax.lax.dot_general(
        ht, w2_ref[...],
        (((0,), (0,)), ((), ())),
        preferred_element_type=jnp.float32)
    o = o.astype(jnp.bfloat16).reshape(G, N, OUT_DIM)
    half = N // 2
    while half >= 8:
        o = jnp.maximum(o[:, :half, :], o[:, half:, :])
        half //= 2
    m = jnp.max(o, axis=1).astype(jnp.float32)
    out_ref[...] = m + b2_ref[...]


def kernel(point_groups, W1, b1, W2, b2):
    num_patches = B * P
    xt = jnp.swapaxes(point_groups, -1, -2)
    xt = xt.reshape(num_patches, IN_DIM, N)
    xt = jnp.transpose(xt, (1, 0, 2)).reshape(IN_DIM, num_patches * N)
    xt = xt.astype(jnp.bfloat16)
    ones = jnp.ones((1, num_patches * N), dtype=jnp.bfloat16)
    xa = jnp.concatenate([xt, ones], axis=0)
    w1a = jnp.concatenate([W1, b1.reshape(1, HID)], axis=0).T
    grid = (num_patches // G,)
    out = pl.pallas_call(
        _body,
        grid=grid,
        in_specs=[
            pl.BlockSpec((AUG, G * N), lambda i: (0, i)),
            pl.BlockSpec((HID, AUG), lambda i: (0, 0)),
            pl.BlockSpec((HID, OUT_DIM), lambda i: (0, 0)),
            pl.BlockSpec((1, OUT_DIM), lambda i: (0, 0)),
        ],
        out_specs=pl.BlockSpec((G, OUT_DIM), lambda i: (i, 0)),
        out_shape=jax.ShapeDtypeStruct((num_patches, OUT_DIM), jnp.float32),
    )(xa, w1a.astype(jnp.bfloat16), W2.astype(jnp.bfloat16),
      b2.reshape(1, OUT_DIM))
    return out.reshape(B, P, OUT_DIM)


# speedup vs baseline: 1.1225x; 1.0131x over previous
import jax
import jax.numpy as jnp
from jax.experimental import pallas as pl

B, P, N = 8, 512, 128
IN_DIM, HID, OUT_DIM = 3, 64, 60
AUG = IN_DIM + 1
G = 256


def _body(x_ref, w1t_ref, b1_ref, w2_ref, b2_ref, out_ref):
    x = x_ref[...]
    ht = jnp.dot(w1t_ref[...], x,
                 preferred_element_type=jnp.float32) + b1_ref[...]
    ht = jnp.maximum(ht.astype(jnp.bfloat16), 0)
    o = jax.lax.dot_general(
        ht, w2_ref[...],
        (((0,), (0,)), ((), ())),
        preferred_element_type=jnp.float32)
    o = o.astype(jnp.bfloat16).reshape(G, N, OUT_DIM)
    half = N // 2
    while half >= 8:
        o = jnp.maximum(o[:, :half, :], o[:, half:, :])
        half //= 2
    m = jnp.max(o, axis=1).astype(jnp.float32)
    out_ref[...] = m + b2_ref[...]


def kernel(point_groups, W1, b1, W2, b2):
    num_patches = B * P
    xt = jnp.swapaxes(point_groups, -1, -2)
    xt = xt.reshape(num_patches, IN_DIM, N)
    xt = jnp.transpose(xt, (1, 0, 2)).reshape(IN_DIM, num_patches * N)
    xt = xt.astype(jnp.bfloat16)
    grid = (num_patches // G,)
    out = pl.pallas_call(
        _body,
        grid=grid,
        in_specs=[
            pl.BlockSpec((IN_DIM, G * N), lambda i: (0, i)),
            pl.BlockSpec((HID, IN_DIM), lambda i: (0, 0)),
            pl.BlockSpec((HID, 1), lambda i: (0, 0)),
            pl.BlockSpec((HID, OUT_DIM), lambda i: (0, 0)),
            pl.BlockSpec((1, OUT_DIM), lambda i: (0, 0)),
        ],
        out_specs=pl.BlockSpec((G, OUT_DIM), lambda i: (i, 0)),
        out_shape=jax.ShapeDtypeStruct((num_patches, OUT_DIM), jnp.float32),
    )(xt, W1.T.astype(jnp.bfloat16), b1.reshape(HID, 1),
      W2.astype(jnp.bfloat16), b2.reshape(1, OUT_DIM))
    return out.reshape(B, P, OUT_DIM)
